# Initial kernel scaffold; baseline (speedup 1.0000x reference)
import jax
import jax.numpy as jnp
from jax.experimental import pallas as pl


def kernel(user, item, it_in, it_off, u_bias_w, i_bias_w, u_embed_w, i_embed_w, t_embed_w):
    raise NotImplementedError("write your pallas kernel here")


# speedup vs baseline: 3.7109x; 3.7109x over previous
import jax
import jax.numpy as jnp
from jax import lax
from jax.experimental import pallas as pl
from jax.experimental.pallas import tpu as pltpu
from jax.experimental.pallas import tpu_sc as plsc

B = 16384
D = 128
NC, NS, L = 2, 16, 16
NW = NC * NS
BPW = B // NW
CH = 128
NCH = BPW // CH


def _sc_body(user_h, item_h, tag_h, ub_h, ib_h, ue_h, ie_h, te_h, out_h,
             uidx, iidx, tidx, urows, irows, trows, ubv, ibv, scorev, sem):
    wid = lax.axis_index("s") * NC + lax.axis_index("c")
    base = wid * BPW

    def chunk(c, carry0):
        off = base + c * CH
        pltpu.sync_copy(user_h.at[pl.ds(off, CH)], uidx)
        pltpu.sync_copy(item_h.at[pl.ds(off, CH)], iidx)
        pltpu.sync_copy(tag_h.at[pl.ds(off, CH)], tidx)
        pltpu.async_copy(ue_h.at[uidx], urows, sem).wait()
        pltpu.async_copy(ie_h.at[iidx], irows, sem).wait()
        pltpu.async_copy(te_h.at[tidx], trows, sem).wait()
        pltpu.async_copy(ub_h.at[uidx], ubv, sem).wait()
        pltpu.async_copy(ib_h.at[iidx], ibv, sem).wait()

        def group(g, carry):
            rows = g * L + lax.iota(jnp.int32, L)
            acc = jnp.zeros((L,), jnp.float32)
            for d in range(D):
                dsplat = jnp.full((L,), d, jnp.int32)
                gu = plsc.load_gather(urows, [rows, dsplat])
                gi = plsc.load_gather(irows, [rows, dsplat])
                gt = plsc.load_gather(trows, [rows, dsplat])
                acc = acc + gu * (gi + gt)
            scorev[pl.ds(g * L, L)] = acc + ubv[pl.ds(g * L, L)] + ibv[pl.ds(g * L, L)]
            return carry

        lax.fori_loop(0, CH // L, group, 0)
        pltpu.sync_copy(scorev, out_h.at[pl.ds(off, CH)])
        return carry0

    lax.fori_loop(0, NCH, chunk, 0)


def kernel(user, item, it_in, it_off, u_bias_w, i_bias_w, u_embed_w, i_embed_w, t_embed_w):
    del it_off
    ub = u_bias_w.reshape(-1)
    ib = i_bias_w.reshape(-1)
    mesh = plsc.VectorSubcoreMesh(core_axis_name="c", subcore_axis_name="s")
    run = pl.kernel(
        _sc_body,
        out_type=jax.ShapeDtypeStruct((B,), jnp.float32),
        mesh=mesh,
        compiler_params=pltpu.CompilerParams(needs_layout_passes=False),
        scratch_types=[
            pltpu.VMEM((CH,), jnp.int32),
            pltpu.VMEM((CH,), jnp.int32),
            pltpu.VMEM((CH,), jnp.int32),
            pltpu.VMEM((CH, D), jnp.float32),
            pltpu.VMEM((CH, D), jnp.float32),
            pltpu.VMEM((CH, D), jnp.float32),
            pltpu.VMEM((CH,), jnp.float32),
            pltpu.VMEM((CH,), jnp.float32),
            pltpu.VMEM((CH,), jnp.float32),
            pltpu.SemaphoreType.DMA,
        ],
    )
    return run(user, item, it_in, ub, ib, u_embed_w, i_embed_w, t_embed_w)


# speedup vs baseline: 5.3430x; 1.4398x over previous
import jax
import jax.numpy as jnp
from jax import lax
from jax.experimental import pallas as pl
from jax.experimental.pallas import tpu as pltpu
from jax.experimental.pallas import tpu_sc as plsc

B = 16384
D = 128
NC, NS, L = 2, 16, 16
NW = NC * NS
BPW = B // NW
CH = 128
NCH = BPW // CH


def _sc_body(user_h, item_h, tag_h, ub_h, ib_h, ue_h, ie_h, te_h, out_h,
             uidx, iidx, tidx, urows, irows, trows, ubv, ibv, scorev,
             gsem, isem, osem):
    wid = lax.axis_index("s") * NC + lax.axis_index("c")
    base = wid * BPW

    def idx_descs(c, p):
        off = base + c * CH
        return (
            pltpu.make_async_copy(user_h.at[pl.ds(off, CH)], uidx.at[p], isem.at[p]),
            pltpu.make_async_copy(item_h.at[pl.ds(off, CH)], iidx.at[p], isem.at[p]),
            pltpu.make_async_copy(tag_h.at[pl.ds(off, CH)], tidx.at[p], isem.at[p]),
        )

    def gather_descs(p):
        return (
            pltpu.make_async_copy(ue_h.at[uidx.at[p]], urows.at[p], gsem.at[p]),
            pltpu.make_async_copy(ie_h.at[iidx.at[p]], irows.at[p], gsem.at[p]),
            pltpu.make_async_copy(te_h.at[tidx.at[p]], trows.at[p], gsem.at[p]),
            pltpu.make_async_copy(ub_h.at[uidx.at[p]], ubv.at[p], gsem.at[p]),
            pltpu.make_async_copy(ib_h.at[iidx.at[p]], ibv.at[p], gsem.at[p]),
        )

    for d in idx_descs(0, 0):
        d.start()
        d.wait()
    for d in gather_descs(0):
        d.start()
    for d in idx_descs(1, 1):
        d.start()

    for i in range(NCH):
        p = i % 2
        np_ = (i + 1) % 2
        for d in gather_descs(p):
            d.wait()
        if i + 1 < NCH:
            for d in idx_descs(i + 1, np_):
                d.wait()
            for d in gather_descs(np_):
                d.start()
        if i + 2 < NCH:
            for d in idx_descs(i + 2, p):
                d.start()
        if i >= 2:
            pltpu.make_async_copy(
                scorev.at[p], out_h.at[pl.ds(base + (i - 2) * CH, CH)], osem
            ).wait()

        def group(g, carry):
            rows = g * L + lax.iota(jnp.int32, L)

            def dblk(dd, acc):
                for j in range(32):
                    dsplat = jnp.full((L,), 1, jnp.int32) * (dd * 32 + j)
                    gu = plsc.load_gather(urows.at[p], [rows, dsplat])
                    gi = plsc.load_gather(irows.at[p], [rows, dsplat])
                    gt = plsc.load_gather(trows.at[p], [rows, dsplat])
                    acc = acc + gu * (gi + gt)
                return acc

            acc = lax.fori_loop(0, D // 32, dblk, jnp.zeros((L,), jnp.float32))
            scorev[p, pl.ds(g * L, L)] = (
                acc + ubv[p, pl.ds(g * L, L)] + ibv[p, pl.ds(g * L, L)]
            )
            return carry

        lax.fori_loop(0, CH // L, group, 0)
        pltpu.make_async_copy(
            scorev.at[p], out_h.at[pl.ds(base + i * CH, CH)], osem
        ).start()

    for i in (NCH - 2, NCH - 1):
        pltpu.make_async_copy(
            scorev.at[i % 2], out_h.at[pl.ds(base + i * CH, CH)], osem
        ).wait()


def kernel(user, item, it_in, it_off, u_bias_w, i_bias_w, u_embed_w, i_embed_w, t_embed_w):
    del it_off
    ub = u_bias_w.reshape(-1)
    ib = i_bias_w.reshape(-1)
    mesh = plsc.VectorSubcoreMesh(core_axis_name="c", subcore_axis_name="s")
    run = pl.kernel(
        _sc_body,
        out_type=jax.ShapeDtypeStruct((B,), jnp.float32),
        mesh=mesh,
        compiler_params=pltpu.CompilerParams(needs_layout_passes=False),
        scratch_types=[
            pltpu.VMEM((2, CH), jnp.int32),
            pltpu.VMEM((2, CH), jnp.int32),
            pltpu.VMEM((2, CH), jnp.int32),
            pltpu.VMEM((2, CH, D), jnp.float32),
            pltpu.VMEM((2, CH, D), jnp.float32),
            pltpu.VMEM((2, CH, D), jnp.float32),
            pltpu.VMEM((2, CH), jnp.float32),
            pltpu.VMEM((2, CH), jnp.float32),
            pltpu.VMEM((2, CH), jnp.float32),
            pltpu.SemaphoreType.DMA((2,)),
            pltpu.SemaphoreType.DMA((2,)),
            pltpu.SemaphoreType.DMA,
        ],
    )
    return run(user, item, it_in, ub, ib, u_embed_w, i_embed_w, t_embed_w)


# speedup vs baseline: 5.3441x; 1.0002x over previous
import jax
import jax.numpy as jnp
from jax import lax
from jax.experimental import pallas as pl
from jax.experimental.pallas import tpu as pltpu
from jax.experimental.pallas import tpu_sc as plsc

B = 16384
D = 128
NC, NS, L = 2, 16, 16
NW = NC * NS
BPW = B // NW
CH = 128
NCH = BPW // CH


def _sc_body(user_h, item_h, tag_h, ub_h, ib_h, ue_h, ie_h, te_h, out_h,
             uidx, iidx, tidx, urows, irows, trows, ubv, ibv, scorev,
             gsem, isem, osem):
    wid = lax.axis_index("s") * NC + lax.axis_index("c")
    base = wid * BPW

    def idx_descs(c, p):
        off = base + c * CH
        return (
            pltpu.make_async_copy(user_h.at[pl.ds(off, CH)], uidx.at[p], isem.at[p]),
            pltpu.make_async_copy(item_h.at[pl.ds(off, CH)], iidx.at[p], isem.at[p]),
            pltpu.make_async_copy(tag_h.at[pl.ds(off, CH)], tidx.at[p], isem.at[p]),
        )

    def gather_descs(p):
        return (
            pltpu.make_async_copy(ue_h.at[uidx.at[p]], urows.at[p], gsem.at[p]),
            pltpu.make_async_copy(ie_h.at[iidx.at[p]], irows.at[p], gsem.at[p]),
            pltpu.make_async_copy(te_h.at[tidx.at[p]], trows.at[p], gsem.at[p]),
            pltpu.make_async_copy(ub_h.at[uidx.at[p]], ubv.at[p], gsem.at[p]),
            pltpu.make_async_copy(ib_h.at[iidx.at[p]], ibv.at[p], gsem.at[p]),
        )

    for d in idx_descs(0, 0):
        d.start()
        d.wait()
    for d in gather_descs(0):
        d.start()
    for d in idx_descs(1, 1):
        d.start()

    for i in range(NCH):
        p = i % 2
        np_ = (i + 1) % 2
        with jax.named_scope("drain_gathers"):
            for d in gather_descs(p):
                d.wait()
        with jax.named_scope("fire_next"):
            if i + 1 < NCH:
                for d in idx_descs(i + 1, np_):
                    d.wait()
                for d in gather_descs(np_):
                    d.start()
            if i + 2 < NCH:
                for d in idx_descs(i + 2, p):
                    d.start()
            if i >= 2:
                pltpu.make_async_copy(
                    scorev.at[p], out_h.at[pl.ds(base + (i - 2) * CH, CH)], osem
                ).wait()

        def group(g, carry):
            rows = g * L + lax.iota(jnp.int32, L)

            def dblk(dd, acc):
                for j in range(32):
                    dsplat = jnp.full((L,), 1, jnp.int32) * (dd * 32 + j)
                    gu = plsc.load_gather(urows.at[p], [rows, dsplat])
                    gi = plsc.load_gather(irows.at[p], [rows, dsplat])
                    gt = plsc.load_gather(trows.at[p], [rows, dsplat])
                    acc = acc + gu * (gi + gt)
                return acc

            acc = lax.fori_loop(0, D // 32, dblk, jnp.zeros((L,), jnp.float32))
            scorev[p, pl.ds(g * L, L)] = (
                acc + ubv[p, pl.ds(g * L, L)] + ibv[p, pl.ds(g * L, L)]
            )
            return carry

        with jax.named_scope("compute"):
            lax.fori_loop(0, CH // L, group, 0)
        pltpu.make_async_copy(
            scorev.at[p], out_h.at[pl.ds(base + i * CH, CH)], osem
        ).start()

    for i in (NCH - 2, NCH - 1):
        pltpu.make_async_copy(
            scorev.at[i % 2], out_h.at[pl.ds(base + i * CH, CH)], osem
        ).wait()


def kernel(user, item, it_in, it_off, u_bias_w, i_bias_w, u_embed_w, i_embed_w, t_embed_w):
    del it_off
    ub = u_bias_w.reshape(-1)
    ib = i_bias_w.reshape(-1)
    mesh = plsc.VectorSubcoreMesh(core_axis_name="c", subcore_axis_name="s")
    run = pl.kernel(
        _sc_body,
        out_type=jax.ShapeDtypeStruct((B,), jnp.float32),
        mesh=mesh,
        compiler_params=pltpu.CompilerParams(needs_layout_passes=False),
        scratch_types=[
            pltpu.VMEM((2, CH), jnp.int32),
            pltpu.VMEM((2, CH), jnp.int32),
            pltpu.VMEM((2, CH), jnp.int32),
            pltpu.VMEM((2, CH, D), jnp.float32),
            pltpu.VMEM((2, CH, D), jnp.float32),
            pltpu.VMEM((2, CH, D), jnp.float32),
            pltpu.VMEM((2, CH), jnp.float32),
            pltpu.VMEM((2, CH), jnp.float32),
            pltpu.VMEM((2, CH), jnp.float32),
            pltpu.SemaphoreType.DMA((2,)),
            pltpu.SemaphoreType.DMA((2,)),
            pltpu.SemaphoreType.DMA,
        ],
    )
    return run(user, item, it_in, ub, ib, u_embed_w, i_embed_w, t_embed_w)


# speedup vs baseline: 15.9230x; 2.9795x over previous
import jax
import jax.numpy as jnp
from jax import lax
from jax.experimental import pallas as pl
from jax.experimental.pallas import tpu as pltpu
from jax.experimental.pallas import tpu_sc as plsc

B = 16384
D = 128
NC, NS, L = 2, 16, 16
NW = NC * NS
BPW = B // NW
CH = 128
NCH = BPW // CH


def _sc_body(user_h, item_h, tag_h, ub_h, ib_h, ue_h, ie_h, te_h, out_h,
             uidx, iidx, tidx, urows, irows, trows, ubv, ibv, scorev,
             gsem, isem, osem):
    wid = lax.axis_index("s") * NC + lax.axis_index("c")
    base = wid * BPW

    def idx_descs(c, p):
        off = base + c * CH
        return (
            pltpu.make_async_copy(user_h.at[pl.ds(off, CH)], uidx.at[p], isem.at[p]),
            pltpu.make_async_copy(item_h.at[pl.ds(off, CH)], iidx.at[p], isem.at[p]),
            pltpu.make_async_copy(tag_h.at[pl.ds(off, CH)], tidx.at[p], isem.at[p]),
        )

    def gather_descs(p):
        return (
            pltpu.make_async_copy(ue_h.at[uidx.at[p]], urows.at[p], gsem.at[p]),
            pltpu.make_async_copy(ie_h.at[iidx.at[p]], irows.at[p], gsem.at[p]),
            pltpu.make_async_copy(te_h.at[tidx.at[p]], trows.at[p], gsem.at[p]),
            pltpu.make_async_copy(ub_h.at[uidx.at[p]], ubv.at[p], gsem.at[p]),
            pltpu.make_async_copy(ib_h.at[iidx.at[p]], ibv.at[p], gsem.at[p]),
        )

    for d in idx_descs(0, 0):
        d.start()
        d.wait()
    for d in gather_descs(0):
        d.start()
    for d in idx_descs(1, 1):
        d.start()

    for i in range(NCH):
        p = i % 2
        np_ = (i + 1) % 2
        with jax.named_scope("drain_gathers"):
            for d in gather_descs(p):
                d.wait()
        with jax.named_scope("fire_next"):
            if i + 1 < NCH:
                for d in idx_descs(i + 1, np_):
                    d.wait()
                for d in gather_descs(np_):
                    d.start()
            if i + 2 < NCH:
                for d in idx_descs(i + 2, p):
                    d.start()
            if i >= 2:
                pltpu.make_async_copy(
                    scorev.at[p], out_h.at[pl.ds(base + (i - 2) * CH, CH)], osem
                ).wait()

        def group(g, carry):
            lane = lax.iota(jnp.int32, L)

            def row(rr, acc):
                r = g * L + rr
                dv = urows[p, r, pl.ds(0, L)] * (
                    irows[p, r, pl.ds(0, L)] + trows[p, r, pl.ds(0, L)])
                for k in range(1, D // L):
                    dv = dv + urows[p, r, pl.ds(k * L, L)] * (
                        irows[p, r, pl.ds(k * L, L)] + trows[p, r, pl.ds(k * L, L)])
                return jnp.where(lane == rr, jnp.sum(dv), acc)

            acc = lax.fori_loop(0, L, row, jnp.zeros((L,), jnp.float32))
            scorev[p, pl.ds(g * L, L)] = (
                acc + ubv[p, pl.ds(g * L, L)] + ibv[p, pl.ds(g * L, L)]
            )
            return carry

        with jax.named_scope("compute"):
            lax.fori_loop(0, CH // L, group, 0)
        pltpu.make_async_copy(
            scorev.at[p], out_h.at[pl.ds(base + i * CH, CH)], osem
        ).start()

    for i in (NCH - 2, NCH - 1):
        pltpu.make_async_copy(
            scorev.at[i % 2], out_h.at[pl.ds(base + i * CH, CH)], osem
        ).wait()


def kernel(user, item, it_in, it_off, u_bias_w, i_bias_w, u_embed_w, i_embed_w, t_embed_w):
    del it_off
    ub = u_bias_w.reshape(-1)
    ib = i_bias_w.reshape(-1)
    mesh = plsc.VectorSubcoreMesh(core_axis_name="c", subcore_axis_name="s")
    run = pl.kernel(
        _sc_body,
        out_type=jax.ShapeDtypeStruct((B,), jnp.float32),
        mesh=mesh,
        compiler_params=pltpu.CompilerParams(needs_layout_passes=False),
        scratch_types=[
            pltpu.VMEM((2, CH), jnp.int32),
            pltpu.VMEM((2, CH), jnp.int32),
            pltpu.VMEM((2, CH), jnp.int32),
            pltpu.VMEM((2, CH, D), jnp.float32),
            pltpu.VMEM((2, CH, D), jnp.float32),
            pltpu.VMEM((2, CH, D), jnp.float32),
            pltpu.VMEM((2, CH), jnp.float32),
            pltpu.VMEM((2, CH), jnp.float32),
            pltpu.VMEM((2, CH), jnp.float32),
            pltpu.SemaphoreType.DMA((2,)),
            pltpu.SemaphoreType.DMA((2,)),
            pltpu.SemaphoreType.DMA,
        ],
    )
    return run(user, item, it_in, ub, ib, u_embed_w, i_embed_w, t_embed_w)
